# baseline (speedup 1.0000x reference)
import jax
import jax.numpy as jnp
from jax import lax
from jax.experimental import pallas as pl
from jax.experimental.pallas import tpu as pltpu
from jax.experimental.pallas import tpu_sc as plsc

_N = 8388608
_NB = 100000
_NBP = 100352
_NC = 2
_NS = 16
_NW = _NC * _NS
_PER_TILE = _N // _NW
_CH = 4096
_NCHUNK = _PER_TILE // _CH


def _sc_hist_body(inds_hbm, out_hbm, idx_a, idx_b, idx_c, idx_d, idx_e,
                  hist, sem_a, sem_b, sem_c, sem_d, sem_e):
    cid = lax.axis_index("c")
    sid = lax.axis_index("s")
    wid = cid * _NS + sid
    base = wid * _PER_TILE

    zeros = jnp.zeros((16,), jnp.int32)
    ones = jnp.full((16,), 1, jnp.int32)

    def scatter_chunk(idx_ref):
        def body(r, carry):
            ivs = [idx_ref[pl.ds((r * 8 + k) * 16, 16)] for k in range(8)]
            for iv in ivs:
                plsc.addupdate_scatter(hist, [iv], ones)
            return carry
        lax.fori_loop(0, _CH // 128, body, 0, unroll=4)

    bufs = (idx_a, idx_b, idx_c, idx_d, idx_e)
    sems = (sem_a, sem_b, sem_c, sem_d, sem_e)
    nbuf = len(bufs)

    def issue(c):
        return pltpu.async_copy(
            inds_hbm.at[pl.ds(base + c * _CH, _CH)], bufs[c % nbuf],
            sems[c % nbuf])

    for c in range(nbuf - 1):
        issue(c)

    def zero_body(i, carry):
        hist[pl.ds(i * 16, 16)] = zeros
        return carry
    lax.fori_loop(0, _NBP // 16, zero_body, 0, unroll=8)

    ngroups = _NCHUNK // nbuf
    ntail = _NCHUNK - ngroups * nbuf

    def group_body(g, carry):
        for j in range(nbuf):
            c = g * nbuf + j
            jp = (j + nbuf - 1) % nbuf

            @pl.when(c + nbuf - 1 < _NCHUNK)
            def _prefetch():
                pltpu.async_copy(
                    inds_hbm.at[pl.ds(base + (c + nbuf - 1) * _CH, _CH)],
                    bufs[jp], sems[jp])

            pltpu.make_async_copy(
                inds_hbm.at[pl.ds(base, _CH)], bufs[j], sems[j]).wait()
            scatter_chunk(bufs[j])
        return carry
    lax.fori_loop(0, ngroups, group_body, 0, unroll=1)

    for c in range(ngroups * nbuf, _NCHUNK):
        j = c % nbuf
        pltpu.make_async_copy(
            inds_hbm.at[pl.ds(base, _CH)], bufs[j], sems[j]).wait()
        scatter_chunk(bufs[j])

    pltpu.sync_copy(hist, out_hbm.at[pl.ds(wid * _NBP, _NBP)])


_sc_hist = pl.kernel(
    _sc_hist_body,
    out_type=jax.ShapeDtypeStruct((_NW * _NBP,), jnp.int32),
    mesh=plsc.VectorSubcoreMesh(core_axis_name="c", subcore_axis_name="s"),
    scratch_types=[
        pltpu.VMEM((_CH,), jnp.int32),
        pltpu.VMEM((_CH,), jnp.int32),
        pltpu.VMEM((_CH,), jnp.int32),
        pltpu.VMEM((_CH,), jnp.int32),
        pltpu.VMEM((_CH,), jnp.int32),
        pltpu.VMEM((_NBP,), jnp.int32),
        pltpu.SemaphoreType.DMA,
        pltpu.SemaphoreType.DMA,
        pltpu.SemaphoreType.DMA,
        pltpu.SemaphoreType.DMA,
        pltpu.SemaphoreType.DMA,
    ],
    compiler_params=pltpu.CompilerParams(needs_layout_passes=False, skip_device_barrier=True),
)


def _tc_merge_body(h_ref, cnt_ref, lim_ref, s_ref, ss_ref):
    h = jnp.sum(h_ref[...], axis=0)
    r = lax.broadcasted_iota(jnp.int32, (_NBP // 128, 128), 0)
    c = lax.broadcasted_iota(jnp.int32, (_NBP // 128, 128), 1)
    b = r * 128 + c
    cnt_ref[...] = h.reshape(-1)[:_NB]
    lim_ref[...] = b.reshape(-1)[:_NB + 1]
    s_ref[0, 0] = jnp.sum(h * b)
    ss_ref[0, 0] = jnp.sum(h * (b * b))


_tc_merge = pl.pallas_call(
    _tc_merge_body,
    out_shape=[
        jax.ShapeDtypeStruct((_NB,), jnp.int32),
        jax.ShapeDtypeStruct((_NB + 1,), jnp.int32),
        jax.ShapeDtypeStruct((1, 1), jnp.int32),
        jax.ShapeDtypeStruct((1, 1), jnp.int32),
    ],
    out_specs=[
        pl.BlockSpec(memory_space=pltpu.VMEM),
        pl.BlockSpec(memory_space=pltpu.VMEM),
        pl.BlockSpec(memory_space=pltpu.SMEM),
        pl.BlockSpec(memory_space=pltpu.SMEM),
    ],
)


def kernel(inds, num_bins):
    parts = _sc_hist(inds)
    h3 = parts.reshape(_NW, _NBP // 128, 128)
    counts, limits, s, ss = _tc_merge(h3)
    hist_min = jnp.asarray(0, jnp.int32)
    hist_max = jnp.asarray(num_bins - 1, jnp.int32)
    num = jnp.asarray(_N, jnp.int32)
    return (hist_min, hist_max, num, s[0, 0], ss[0, 0], limits, counts)


# speedup vs baseline: 1.0203x; 1.0203x over previous
import jax
import jax.numpy as jnp
from jax import lax
from jax.experimental import pallas as pl
from jax.experimental.pallas import tpu as pltpu
from jax.experimental.pallas import tpu_sc as plsc

_N = 8388608
_NB = 100000
_NBP = 100352
_NC = 2
_NS = 16
_NW = _NC * _NS
_PER_TILE = _N // _NW
_CH = 4096
_NCHUNK = _PER_TILE // _CH


def _sc_hist_body(inds_hbm, out_hbm, idx_a, idx_b, idx_c, idx_d, idx_e,
                  hist, sem_a, sem_b, sem_c, sem_d, sem_e):
    cid = lax.axis_index("c")
    sid = lax.axis_index("s")
    wid = cid * _NS + sid
    base = wid * _PER_TILE

    zeros = jnp.zeros((16,), jnp.int32)
    ones = jnp.full((16,), 1, jnp.int32)

    def scatter_chunk(idx_ref):
        def body(r, carry):
            ivs = [idx_ref[pl.ds((r * 8 + k) * 16, 16)] for k in range(8)]
            for iv in ivs:
                plsc.addupdate_scatter(hist, [iv], ones)
            return carry
        lax.fori_loop(0, _CH // 128, body, 0, unroll=2)

    bufs = (idx_a, idx_b, idx_c, idx_d, idx_e)
    sems = (sem_a, sem_b, sem_c, sem_d, sem_e)
    nbuf = len(bufs)

    def issue(c):
        return pltpu.async_copy(
            inds_hbm.at[pl.ds(base + c * _CH, _CH)], bufs[c % nbuf],
            sems[c % nbuf])

    for c in range(nbuf - 1):
        issue(c)

    def zero_body(i, carry):
        hist[pl.ds(i * 16, 16)] = zeros
        return carry
    lax.fori_loop(0, _NBP // 16, zero_body, 0, unroll=8)

    ngroups = _NCHUNK // nbuf
    ntail = _NCHUNK - ngroups * nbuf

    def group_body(g, carry):
        for j in range(nbuf):
            c = g * nbuf + j
            jp = (j + nbuf - 1) % nbuf

            @pl.when(c + nbuf - 1 < _NCHUNK)
            def _prefetch():
                pltpu.async_copy(
                    inds_hbm.at[pl.ds(base + (c + nbuf - 1) * _CH, _CH)],
                    bufs[jp], sems[jp])

            pltpu.make_async_copy(
                inds_hbm.at[pl.ds(base, _CH)], bufs[j], sems[j]).wait()
            scatter_chunk(bufs[j])
        return carry
    lax.fori_loop(0, ngroups, group_body, 0, unroll=1)

    for c in range(ngroups * nbuf, _NCHUNK):
        j = c % nbuf
        pltpu.make_async_copy(
            inds_hbm.at[pl.ds(base, _CH)], bufs[j], sems[j]).wait()
        scatter_chunk(bufs[j])

    pltpu.sync_copy(hist, out_hbm.at[pl.ds(wid * _NBP, _NBP)])


_sc_hist = pl.kernel(
    _sc_hist_body,
    out_type=jax.ShapeDtypeStruct((_NW * _NBP,), jnp.int32),
    mesh=plsc.VectorSubcoreMesh(core_axis_name="c", subcore_axis_name="s"),
    scratch_types=[
        pltpu.VMEM((_CH,), jnp.int32),
        pltpu.VMEM((_CH,), jnp.int32),
        pltpu.VMEM((_CH,), jnp.int32),
        pltpu.VMEM((_CH,), jnp.int32),
        pltpu.VMEM((_CH,), jnp.int32),
        pltpu.VMEM((_NBP,), jnp.int32),
        pltpu.SemaphoreType.DMA,
        pltpu.SemaphoreType.DMA,
        pltpu.SemaphoreType.DMA,
        pltpu.SemaphoreType.DMA,
        pltpu.SemaphoreType.DMA,
    ],
    compiler_params=pltpu.CompilerParams(needs_layout_passes=False, skip_device_barrier=True),
)


def _tc_merge_body(h_ref, cnt_ref, lim_ref, s_ref, ss_ref):
    h = jnp.sum(h_ref[...], axis=0)
    r = lax.broadcasted_iota(jnp.int32, (_NBP // 128, 128), 0)
    c = lax.broadcasted_iota(jnp.int32, (_NBP // 128, 128), 1)
    b = r * 128 + c
    cnt_ref[...] = h.reshape(-1)[:_NB]
    lim_ref[...] = b.reshape(-1)[:_NB + 1]
    s_ref[0, 0] = jnp.sum(h * b)
    ss_ref[0, 0] = jnp.sum(h * (b * b))


_tc_merge = pl.pallas_call(
    _tc_merge_body,
    out_shape=[
        jax.ShapeDtypeStruct((_NB,), jnp.int32),
        jax.ShapeDtypeStruct((_NB + 1,), jnp.int32),
        jax.ShapeDtypeStruct((1, 1), jnp.int32),
        jax.ShapeDtypeStruct((1, 1), jnp.int32),
    ],
    out_specs=[
        pl.BlockSpec(memory_space=pltpu.VMEM),
        pl.BlockSpec(memory_space=pltpu.VMEM),
        pl.BlockSpec(memory_space=pltpu.SMEM),
        pl.BlockSpec(memory_space=pltpu.SMEM),
    ],
)


def kernel(inds, num_bins):
    parts = _sc_hist(inds)
    h3 = parts.reshape(_NW, _NBP // 128, 128)
    counts, limits, s, ss = _tc_merge(h3)
    hist_min = jnp.asarray(0, jnp.int32)
    hist_max = jnp.asarray(num_bins - 1, jnp.int32)
    num = jnp.asarray(_N, jnp.int32)
    return (hist_min, hist_max, num, s[0, 0], ss[0, 0], limits, counts)
